# baseline (speedup 1.0000x reference)
import functools

import jax
import jax.numpy as jnp
from jax import lax
from jax.experimental import pallas as pl
from jax.experimental.pallas import tpu as pltpu
from jax.experimental.pallas import tpu_sc as plsc

NC = 2
NS = 16
NW = NC * NS

EMBED_DIM = 64
PAD_DIM = 128
BC = 128
L = 16
RU = 8


def _make_gather(batch: int, hist: int):
    assert batch % (NW * BC) == 0
    nb = batch // NW
    n_bc = nb // BC
    mesh = plsc.VectorSubcoreMesh(core_axis_name="c", subcore_axis_name="s")

    @functools.partial(
        pl.kernel,
        out_type=jax.ShapeDtypeStruct((hist, batch, EMBED_DIM), jnp.float32),
        mesh=mesh,
        compiler_params=pltpu.CompilerParams(use_tc_tiling_on_sc=True),
        scratch_types=[
            pltpu.VMEM((hist, BC), jnp.int32),
            pltpu.VMEM((BC, PAD_DIM), jnp.float32),
            pltpu.VMEM((BC, PAD_DIM), jnp.float32),
            pltpu.VMEM((BC, PAD_DIM), jnp.float32),
            pltpu.VMEM((BC, EMBED_DIM), jnp.float32),
            pltpu.VMEM((BC, EMBED_DIM), jnp.float32),
            pltpu.SemaphoreType.DMA,
            pltpu.SemaphoreType.DMA,
            pltpu.SemaphoreType.DMA,
            pltpu.SemaphoreType.DMA,
            pltpu.SemaphoreType.DMA,
            pltpu.SemaphoreType.DMA,
        ],
    )
    def gather_kernel(idx_hbm, table_hbm, out_hbm, idx_v, g0, g1, g2, o0, o1,
                      isem, gsem0, gsem1, gsem2, osem0, osem1):
        wid = lax.axis_index("s") * NC + lax.axis_index("c")
        b0 = wid * nb
        gbufs = (g0, g1, g2)
        obufs = (o0, o1)
        gsems = (gsem0, gsem1, gsem2)
        osems = (osem0, osem1)

        @pl.loop(0, n_bc)
        def _bchunk(bc):
            bstart = b0 + bc * BC
            pltpu.async_copy(
                idx_hbm.at[:, pl.ds(bstart, BC)], idx_v, isem,
            ).wait()
            pltpu.async_copy(table_hbm.at[idx_v.at[0]], g0, gsem0)
            pltpu.async_copy(table_hbm.at[idx_v.at[1]], g1, gsem1)
            for h in range(hist):
                k = h % 3
                ko = h % 2
                pltpu.make_async_copy(
                    table_hbm.at[idx_v.at[h]], gbufs[k], gsems[k]
                ).wait()
                if h + 2 < hist:
                    pltpu.async_copy(
                        table_hbm.at[idx_v.at[h + 2]], gbufs[(h + 2) % 3],
                        gsems[(h + 2) % 3],
                    )
                if h >= 2:
                    pltpu.make_async_copy(
                        obufs[ko], out_hbm.at[0, pl.ds(0, BC), :], osems[ko]
                    ).wait()
                else:
                    @pl.when(bc > 0)
                    def _():
                        pltpu.make_async_copy(
                            obufs[ko], out_hbm.at[0, pl.ds(0, BC), :],
                            osems[ko],
                        ).wait()
                gb, ob = gbufs[k], obufs[ko]

                @pl.loop(0, BC, step=RU)
                def _rows(r):
                    for rr in range(RU):
                        for v in range(EMBED_DIM // L):
                            ob[r + rr, pl.ds(v * L, L)] = (
                                gb[r + rr, pl.ds(v * L, L)])

                pltpu.async_copy(
                    ob, out_hbm.at[h, pl.ds(bstart, BC), :], osems[ko]
                )

        for b in range(2):
            pltpu.make_async_copy(
                obufs[b], out_hbm.at[0, pl.ds(0, BC), :], osems[b]
            ).wait()

    return gather_kernel


def kernel(inp, weight):
    batch, hist = inp.shape
    idx_t = jnp.transpose(inp).astype(jnp.int32)
    table = jnp.pad(weight, ((0, 0), (0, PAD_DIM - EMBED_DIM)))
    out = _make_gather(batch, hist)(idx_t, table)
    return jnp.transpose(out, (1, 0, 2))
